# baseline (speedup 1.0000x reference)
import functools

import jax
import jax.numpy as jnp
from jax import lax
from jax.experimental import pallas as pl
from jax.experimental.pallas import tpu as pltpu
from jax.experimental.pallas import tpu_sc as plsc

_N = 10000
_E = 160000
_HC = 1024
_H = 4
_G = 64
_NT = 32
_KCH = 32
_NCHK = _HC // 16
_NPB = 312
_EPS = 1e-16
_NEG = -1e30


def _sread(ref, i):
    return ref[pl.ds(i, 1)][0]


def _dyng(v, idx):
    return lax.gather(
        v, idx[:, None],
        dimension_numbers=lax.GatherDimensionNumbers(
            offset_dims=(), collapsed_slice_dims=(0,), start_index_map=(0,)),
        slice_sizes=(1,), mode=lax.GatherScatterMode.PROMISE_IN_BOUNDS)


def _allsum(v):
    iota = lax.iota(jnp.int32, 16)
    for k in (1, 2, 4, 8):
        v = v + _dyng(v, jnp.bitwise_xor(iota, k))
    return v


def _lo2f(w):
    return lax.bitcast_convert_type(w << 16, jnp.float32)


def _hi2f(w):
    return lax.bitcast_convert_type(w & jnp.int32(-65536), jnp.float32)


def _gat_body(pool, xl_hbm, xr_hbm, ssrc_hbm, sdst_hbm, sea_hbm, est_hbm,
              wev_hbm, attv_hbm, biasv_hbm, batch_hbm, out_hbm,
              est_v, wev_v, attv_v, biasv_v, batch_v,
              src_c0, src_c1, dst_c0, dst_c1, ea_c0, ea_c1,
              lrows0, lrows1, rrows0, rrows1,
              acc_v, stage_v, negrow_v, gmax_v, s_v,
              msem0, msem1, gsem0, gsem1):
    src_c = (src_c0, src_c1)
    dst_c = (dst_c0, dst_c1)
    ea_c = (ea_c0, ea_c1)
    lrows = (lrows0, lrows1)
    rrows = (rrows0, rrows1)
    msem = (msem0, msem1)
    gsem = (gsem0, gsem1)

    wid = lax.axis_index("c") * 16 + lax.axis_index("s")
    n0 = wid * _NPB
    n1 = jnp.where(wid == _NT - 1, _N, (wid + 1) * _NPB)
    pltpu.sync_copy(est_hbm, est_v)
    pltpu.sync_copy(wev_hbm, wev_v)
    pltpu.sync_copy(attv_hbm, attv_v)
    pltpu.sync_copy(biasv_hbm, biasv_v)
    e_lo = _sread(est_v, wid)
    e_hi = _sread(est_v, wid + 1)

    zero16 = jnp.zeros((16,), jnp.float32)
    neg16 = jnp.full((16,), _NEG, jnp.float32)

    nb = jnp.minimum((n0 // 8) * 8, _N - 328)
    if pool:
        pltpu.sync_copy(batch_hbm.at[pl.ds(nb, 328)], batch_v)
        for k in range(_NCHK):
            negrow_v[pl.ds(k * 16, 16)] = neg16
            gmax_v[pl.ds(k * 16, 16)] = neg16
        def _initg(g, c):
            off = pl.multiple_of((wid * _G + g) * _HC, _HC)
            pltpu.sync_copy(negrow_v, out_hbm.at[pl.ds(off, _HC)])
            return c
        lax.fori_loop(0, _G, _initg, 0)
        aux0 = _sread(batch_v, n0 - nb)
    else:
        aux0 = n0

    for k in range(_NCHK):
        acc_v[pl.ds(k * 16, 16)] = zero16
    for h in range(_H):
        s_v[pl.ds(h * 16, 16)] = zero16

    def emit(cur, aux):
        recips = [1.0 / (s_v[pl.ds(h * 16, 16)] + _EPS) for h in range(_H)]
        for h in range(_H):
            s_v[pl.ds(h * 16, 16)] = zero16
        if pool:
            g = _sread(batch_v, cur - nb)
            def _flushg(c):
                off = pl.multiple_of((wid * _G + aux) * _HC, _HC)
                pltpu.sync_copy(gmax_v, out_hbm.at[pl.ds(off, _HC)])
                for k in range(_NCHK):
                    gmax_v[pl.ds(k * 16, 16)] = neg16
                return c
            _ = lax.cond(g != aux, _flushg, lambda c: c, 0)
            for k in range(_NCHK):
                sl = pl.ds(k * 16, 16)
                ov = acc_v[sl] * recips[k // 16] + biasv_v[sl]
                ov = jnp.where(ov > 0, ov, jnp.exp(ov) - 1.0)
                gmax_v[sl] = jnp.maximum(gmax_v[sl], ov)
                acc_v[sl] = zero16
            return g
        else:
            sb = aux
            slot = cur - sb
            for k in range(_NCHK):
                sl = pl.ds(k * 16, 16)
                ov = acc_v[sl] * recips[k // 16] + biasv_v[sl]
                ov = jnp.where(ov > 0, ov, jnp.exp(ov) - 1.0)
                stage_v[slot, sl] = ov
                acc_v[sl] = zero16
            def _flush(c):
                sba = pl.multiple_of(sb, 8)
                pltpu.sync_copy(stage_v, out_hbm.at[pl.ds(sba, 8)])
                return c
            _ = lax.cond(slot == 7, _flush, lambda c: c, 0)
            return jnp.where(slot == 7, sb + 8, sb)

    def finalize_to(target, carry):
        cur, aux = carry
        def _fb(i, a):
            return emit(cur + i, a)
        aux = lax.fori_loop(0, jnp.maximum(target - cur, 0), _fb, aux)
        return (jnp.maximum(cur, target), aux)

    def process(b, j, carry):
        d = _sread(dst_c[b], j)
        cur, aux = finalize_to(d, carry)
        eav = _sread(ea_c[b], j)
        pa = [zero16, zero16, zero16, zero16]
        for g in range(_NCHK // 2):
            slw = pl.ds(g * 16, 16)
            xlw = lrows[b][j, slw]
            xrw = rrows[b][j, slw]
            xl_eo = (_lo2f(xlw), _hi2f(xlw))
            xr_eo = (_lo2f(xrw), _hi2f(xrw))
            for half in (0, 1):
                k = 2 * g + half
                sl = pl.ds(k * 16, 16)
                v = xl_eo[half] + xr_eo[half] + eav * wev_v[sl]
                v = jnp.maximum(v, 0.2 * v)
                pa[k // 16] = pa[k // 16] + v * attv_v[sl]
        w = [jnp.exp(_allsum(p)) for p in pa]
        for g in range(_NCHK // 2):
            slw = pl.ds(g * 16, 16)
            xlw = lrows[b][j, slw]
            xl_eo = (_lo2f(xlw), _hi2f(xlw))
            for half in (0, 1):
                k = 2 * g + half
                sl = pl.ds(k * 16, 16)
                acc_v[sl] = acc_v[sl] + w[k // 16] * xl_eo[half]
        for h in range(_H):
            sl = pl.ds(h * 16, 16)
            s_v[sl] = s_v[sl] + w[h]
        return (d, aux)

    c0 = (e_lo // _KCH) * _KCH
    nch = jnp.maximum((e_hi - c0 + _KCH - 1) // _KCH, 0)

    def _bofs(ci):
        return c0 + ci * _KCH

    def meta_issue(base, b):
        pltpu.async_copy(ssrc_hbm.at[pl.ds(base, _KCH)], src_c[b], msem[b])
        pltpu.async_copy(sdst_hbm.at[pl.ds(base, _KCH)], dst_c[b], msem[b])
        pltpu.async_copy(sea_hbm.at[pl.ds(base, _KCH)], ea_c[b], msem[b])

    def meta_wait(base, b):
        pltpu.make_async_copy(ssrc_hbm.at[pl.ds(base, _KCH)], src_c[b],
                              msem[b]).wait()
        pltpu.make_async_copy(sdst_hbm.at[pl.ds(base, _KCH)], dst_c[b],
                              msem[b]).wait()
        pltpu.make_async_copy(sea_hbm.at[pl.ds(base, _KCH)], ea_c[b],
                              msem[b]).wait()

    def gather_issue(b):
        pltpu.async_copy(xl_hbm.at[src_c[b]], lrows[b], gsem[b])
        pltpu.async_copy(xr_hbm.at[dst_c[b]], rrows[b], gsem[b])

    def gather_wait(b):
        pltpu.make_async_copy(xl_hbm.at[src_c[b]], lrows[b], gsem[b]).wait()
        pltpu.make_async_copy(xr_hbm.at[dst_c[b]], rrows[b], gsem[b]).wait()

    @pl.when(nch > 0)
    def _prolog():
        meta_issue(c0, 0)
        meta_wait(c0, 0)
        gather_issue(0)

    @pl.when(nch > 1)
    def _prolog2():
        meta_issue(_bofs(1), 1)

    def pair(cp, carry):
        for b in (0, 1):
            ci = 2 * cp + b
            ob = 1 - b

            @pl.when(ci + 1 < nch)
            def _():
                meta_wait(_bofs(ci + 1), ob)
                gather_issue(ob)

            def _comp(c, b=b, ci=ci):
                gather_wait(b)
                base = _bofs(ci)
                j_lo = jnp.maximum(e_lo - base, 0)
                j_hi = jnp.minimum(e_hi - base, _KCH)
                return lax.fori_loop(j_lo, j_hi,
                                     functools.partial(process, b), c)

            carry = lax.cond(ci < nch, _comp, lambda c: c, carry)

            @pl.when(ci + 2 < nch)
            def _():
                meta_issue(_bofs(ci + 2), b)
        return carry

    carry = lax.fori_loop(0, (nch + 1) // 2, pair, (n0, aux0))
    cur, aux = finalize_to(n1, carry)
    if pool:
        off = pl.multiple_of((wid * _G + aux) * _HC, _HC)
        pltpu.sync_copy(gmax_v, out_hbm.at[pl.ds(off, _HC)])


def _run_gat_sc(pool, xl, xr, ssrc, sdst, sea, est, wev, attv, biasv, batch):
    mesh = plsc.VectorSubcoreMesh(core_axis_name="c", subcore_axis_name="s")
    out_type = (jax.ShapeDtypeStruct((_NT * _G * _HC,), jnp.float32) if pool
                else jax.ShapeDtypeStruct((_N, _HC), jnp.float32))
    scratch = [
        pltpu.VMEM((40,), jnp.int32),
        pltpu.VMEM((_HC,), jnp.float32),
        pltpu.VMEM((_HC,), jnp.float32),
        pltpu.VMEM((_HC,), jnp.float32),
        pltpu.VMEM((328,), jnp.int32),
        pltpu.VMEM((_KCH,), jnp.int32),
        pltpu.VMEM((_KCH,), jnp.int32),
        pltpu.VMEM((_KCH,), jnp.int32),
        pltpu.VMEM((_KCH,), jnp.int32),
        pltpu.VMEM((_KCH,), jnp.float32),
        pltpu.VMEM((_KCH,), jnp.float32),
        pltpu.VMEM((_KCH, _HC // 2), jnp.int32),
        pltpu.VMEM((_KCH, _HC // 2), jnp.int32),
        pltpu.VMEM((_KCH, _HC // 2), jnp.int32),
        pltpu.VMEM((_KCH, _HC // 2), jnp.int32),
        pltpu.VMEM((_HC,), jnp.float32),
        pltpu.VMEM((8, _HC), jnp.float32),
        pltpu.VMEM((_HC,), jnp.float32),
        pltpu.VMEM((_HC,), jnp.float32),
        pltpu.VMEM((64,), jnp.float32),
        pltpu.SemaphoreType.DMA,
        pltpu.SemaphoreType.DMA,
        pltpu.SemaphoreType.DMA,
        pltpu.SemaphoreType.DMA,
    ]
    k = pl.kernel(functools.partial(_gat_body, pool), mesh=mesh,
                  out_type=out_type, scratch_types=scratch)
    return k(xl, xr, ssrc, sdst, sea, est, wev, attv, biasv, batch)


def _mm_dual_body(a_ref, b_ref, bias_ref, l_ref, r_ref):
    a = a_ref[...]
    if a.dtype != jnp.bfloat16:
        a = a.astype(jnp.bfloat16)
    b = b_ref[...]
    l_ref[...] = (jnp.dot(a, b[:, :_HC], preferred_element_type=jnp.float32)
                  + bias_ref[0, :_HC]).astype(jnp.bfloat16)
    r_ref[...] = (jnp.dot(a, b[:, _HC:], preferred_element_type=jnp.float32)
                  + bias_ref[0, _HC:]).astype(jnp.bfloat16)


def _mm_dual(a, w, bias):
    m, kk = a.shape
    blk = 1000
    return pl.pallas_call(
        _mm_dual_body,
        grid=(m // blk,),
        in_specs=[pl.BlockSpec((blk, kk), lambda i: (i, 0)),
                  pl.BlockSpec((kk, 2 * _HC), lambda i: (0, 0)),
                  pl.BlockSpec((1, 2 * _HC), lambda i: (0, 0))],
        out_specs=[pl.BlockSpec((blk, _HC), lambda i: (i, 0)),
                   pl.BlockSpec((blk, _HC), lambda i: (i, 0))],
        out_shape=[jax.ShapeDtypeStruct((m, _HC), jnp.bfloat16),
                   jax.ShapeDtypeStruct((m, _HC), jnp.bfloat16)],
    )(a, w.astype(jnp.bfloat16), bias.reshape(1, -1))


def _pool_mlp_body(p_ref, wp_ref, bp_ref, w1_ref, b1_ref, w2_ref, b2_ref,
                   o_ref):
    gmax = jnp.max(p_ref[...], axis=0)
    gmax = jnp.where(gmax < -1e29, 0.0, gmax)
    p = (jnp.dot(gmax, wp_ref[...], preferred_element_type=jnp.float32)
         + bp_ref[0])
    z = jnp.maximum(
        jnp.dot(p, w1_ref[...], preferred_element_type=jnp.float32)
        + b1_ref[0], 0.0)
    o_ref[...] = (jnp.dot(z, w2_ref[...], preferred_element_type=jnp.float32)
                  + b2_ref[0])


def _pool_mlp(partials, wp, bp, w1, b1, w2, b2):
    return pl.pallas_call(
        _pool_mlp_body,
        out_shape=jax.ShapeDtypeStruct((_G, w2.shape[1]), jnp.float32),
    )(partials, wp, bp.reshape(1, -1), w1, b1.reshape(1, -1), w2,
      b2.reshape(1, -1))


def kernel(x, edge_index, edge_attr, batch, Wl1, bl1, Wr1, br1, We1, att1,
           bias1, Wl2, bl2, Wr2, br2, We2, att2, bias2, Wp, bp, W1, b1, W2,
           b2):
    src = edge_index[0].astype(jnp.int32)
    dst = edge_index[1].astype(jnp.int32)
    ea = edge_attr[:, 0]

    eid = lax.iota(jnp.uint32, _E)
    skey = lax.sort((dst.astype(jnp.uint32) << 18) | eid)
    perm = (skey & jnp.uint32(0x3FFFF)).astype(jnp.int32)
    sdst = (skey >> 18).astype(jnp.int32)
    ssrc = jnp.take(src, perm)
    sea = jnp.take(ea, perm)
    pad = _KCH + 8
    sdst_p = jnp.concatenate([sdst, jnp.zeros((pad,), jnp.int32)])
    ssrc_p = jnp.concatenate([ssrc, jnp.zeros((pad,), jnp.int32)])
    sea_p = jnp.concatenate([sea, jnp.zeros((pad,), jnp.float32)])
    iperm = []
    for g in range(_HC // 32):
        iperm += [g * 32 + 2 * i for i in range(16)]
        iperm += [g * 32 + 2 * i + 1 for i in range(16)]
    iperm = jnp.array(iperm, jnp.int32)
    bounds = jnp.array([t * _NPB for t in range(_NT)] + [_N], jnp.int32)
    est = jnp.concatenate([
        jnp.searchsorted(sdst, bounds, side='left').astype(jnp.int32),
        jnp.zeros((7,), jnp.int32)])
    batch32 = batch.astype(jnp.int32)

    def _pack_rows(m2d):
        return lax.bitcast_convert_type(
            m2d.reshape(m2d.shape[0], _HC // 2, 2), jnp.int32)

    xl1, xr1 = _mm_dual(x, jnp.concatenate([Wl1, Wr1], axis=1),
                        jnp.concatenate([bl1, br1]))
    h1 = _run_gat_sc(False, _pack_rows(xl1), _pack_rows(xr1), ssrc_p,
                     sdst_p, sea_p, est, We1[0].astype(jnp.float32)[iperm],
                     att1.reshape(-1).astype(jnp.float32)[iperm],
                     bias1[iperm], batch32)
    xl2, xr2 = _mm_dual(h1, jnp.concatenate([Wl2, Wr2], axis=1)[iperm],
                        jnp.concatenate([bl2, br2]))
    partials = _run_gat_sc(True, _pack_rows(xl2), _pack_rows(xr2), ssrc_p,
                           sdst_p, sea_p, est, We2[0].astype(jnp.float32)[iperm],
                           att2.reshape(-1).astype(jnp.float32)[iperm],
                           bias2[iperm], batch32)
    return _pool_mlp(partials.reshape(_NT, _G, _HC), Wp[iperm], bp, W1, b1,
                     W2, b2)


# speedup vs baseline: 1.0117x; 1.0117x over previous
import functools

import jax
import jax.numpy as jnp
from jax import lax
from jax.experimental import pallas as pl
from jax.experimental.pallas import tpu as pltpu
from jax.experimental.pallas import tpu_sc as plsc

_N = 10000
_E = 160000
_HC = 1024
_H = 4
_G = 64
_NT = 32
_KCH = 48
_NCHK = _HC // 16
_NPB = 312
_EPS = 1e-16
_NEG = -1e30


def _sread(ref, i):
    return ref[pl.ds(i, 1)][0]


def _dyng(v, idx):
    return lax.gather(
        v, idx[:, None],
        dimension_numbers=lax.GatherDimensionNumbers(
            offset_dims=(), collapsed_slice_dims=(0,), start_index_map=(0,)),
        slice_sizes=(1,), mode=lax.GatherScatterMode.PROMISE_IN_BOUNDS)


def _allsum(v):
    iota = lax.iota(jnp.int32, 16)
    for k in (1, 2, 4, 8):
        v = v + _dyng(v, jnp.bitwise_xor(iota, k))
    return v


def _lo2f(w):
    return lax.bitcast_convert_type(w << 16, jnp.float32)


def _hi2f(w):
    return lax.bitcast_convert_type(w & jnp.int32(-65536), jnp.float32)


def _gat_body(pool, xl_hbm, xr_hbm, ssrc_hbm, sdst_hbm, sea_hbm, est_hbm,
              wev_hbm, attv_hbm, biasv_hbm, batch_hbm, out_hbm,
              est_v, wev_v, attv_v, biasv_v, batch_v,
              src_c0, src_c1, dst_c0, dst_c1, ea_c0, ea_c1,
              lrows0, lrows1, rrows0, rrows1,
              acc_v, stage_v, negrow_v, gmax_v, s_v,
              msem0, msem1, gsem0, gsem1):
    src_c = (src_c0, src_c1)
    dst_c = (dst_c0, dst_c1)
    ea_c = (ea_c0, ea_c1)
    lrows = (lrows0, lrows1)
    rrows = (rrows0, rrows1)
    msem = (msem0, msem1)
    gsem = (gsem0, gsem1)

    wid = lax.axis_index("c") * 16 + lax.axis_index("s")
    n0 = wid * _NPB
    n1 = jnp.where(wid == _NT - 1, _N, (wid + 1) * _NPB)
    pltpu.sync_copy(est_hbm, est_v)
    pltpu.sync_copy(wev_hbm, wev_v)
    pltpu.sync_copy(attv_hbm, attv_v)
    pltpu.sync_copy(biasv_hbm, biasv_v)
    e_lo = _sread(est_v, wid)
    e_hi = _sread(est_v, wid + 1)

    zero16 = jnp.zeros((16,), jnp.float32)
    neg16 = jnp.full((16,), _NEG, jnp.float32)

    nb = jnp.minimum((n0 // 8) * 8, _N - 328)
    if pool:
        pltpu.sync_copy(batch_hbm.at[pl.ds(nb, 328)], batch_v)
        for k in range(_NCHK):
            negrow_v[pl.ds(k * 16, 16)] = neg16
            gmax_v[pl.ds(k * 16, 16)] = neg16
        def _initg(g, c):
            off = pl.multiple_of((wid * _G + g) * _HC, _HC)
            pltpu.sync_copy(negrow_v, out_hbm.at[pl.ds(off, _HC)])
            return c
        lax.fori_loop(0, _G, _initg, 0)
        aux0 = _sread(batch_v, n0 - nb)
    else:
        aux0 = n0

    for k in range(_NCHK):
        acc_v[pl.ds(k * 16, 16)] = zero16
    for h in range(_H):
        s_v[pl.ds(h * 16, 16)] = zero16

    def emit(cur, aux):
        recips = [1.0 / (s_v[pl.ds(h * 16, 16)] + _EPS) for h in range(_H)]
        for h in range(_H):
            s_v[pl.ds(h * 16, 16)] = zero16
        if pool:
            g = _sread(batch_v, cur - nb)
            def _flushg(c):
                off = pl.multiple_of((wid * _G + aux) * _HC, _HC)
                pltpu.sync_copy(gmax_v, out_hbm.at[pl.ds(off, _HC)])
                for k in range(_NCHK):
                    gmax_v[pl.ds(k * 16, 16)] = neg16
                return c
            _ = lax.cond(g != aux, _flushg, lambda c: c, 0)
            for k in range(_NCHK):
                sl = pl.ds(k * 16, 16)
                ov = acc_v[sl] * recips[k // 16] + biasv_v[sl]
                ov = jnp.where(ov > 0, ov, jnp.exp(ov) - 1.0)
                gmax_v[sl] = jnp.maximum(gmax_v[sl], ov)
                acc_v[sl] = zero16
            return g
        else:
            sb = aux
            slot = cur - sb
            for k in range(_NCHK):
                sl = pl.ds(k * 16, 16)
                ov = acc_v[sl] * recips[k // 16] + biasv_v[sl]
                ov = jnp.where(ov > 0, ov, jnp.exp(ov) - 1.0)
                stage_v[slot, sl] = ov
                acc_v[sl] = zero16
            def _flush(c):
                sba = pl.multiple_of(sb, 8)
                pltpu.sync_copy(stage_v, out_hbm.at[pl.ds(sba, 8)])
                return c
            _ = lax.cond(slot == 7, _flush, lambda c: c, 0)
            return jnp.where(slot == 7, sb + 8, sb)

    def finalize_to(target, carry):
        cur, aux = carry
        def _fb(i, a):
            return emit(cur + i, a)
        aux = lax.fori_loop(0, jnp.maximum(target - cur, 0), _fb, aux)
        return (jnp.maximum(cur, target), aux)

    def process(b, j, carry):
        d = _sread(dst_c[b], j)
        cur, aux = finalize_to(d, carry)
        eav = _sread(ea_c[b], j)
        pa = [zero16, zero16, zero16, zero16]
        for g in range(_NCHK // 2):
            slw = pl.ds(g * 16, 16)
            xlw = lrows[b][j, slw]
            xrw = rrows[b][j, slw]
            xl_eo = (_lo2f(xlw), _hi2f(xlw))
            xr_eo = (_lo2f(xrw), _hi2f(xrw))
            for half in (0, 1):
                k = 2 * g + half
                sl = pl.ds(k * 16, 16)
                v = xl_eo[half] + xr_eo[half] + eav * wev_v[sl]
                v = jnp.maximum(v, 0.2 * v)
                pa[k // 16] = pa[k // 16] + v * attv_v[sl]
        w = [jnp.exp(_allsum(p)) for p in pa]
        for g in range(_NCHK // 2):
            slw = pl.ds(g * 16, 16)
            xlw = lrows[b][j, slw]
            xl_eo = (_lo2f(xlw), _hi2f(xlw))
            for half in (0, 1):
                k = 2 * g + half
                sl = pl.ds(k * 16, 16)
                acc_v[sl] = acc_v[sl] + w[k // 16] * xl_eo[half]
        for h in range(_H):
            sl = pl.ds(h * 16, 16)
            s_v[sl] = s_v[sl] + w[h]
        return (d, aux)

    c0 = (e_lo // _KCH) * _KCH
    nch = jnp.maximum((e_hi - c0 + _KCH - 1) // _KCH, 0)

    def _bofs(ci):
        return c0 + ci * _KCH

    def meta_issue(base, b):
        pltpu.async_copy(ssrc_hbm.at[pl.ds(base, _KCH)], src_c[b], msem[b])
        pltpu.async_copy(sdst_hbm.at[pl.ds(base, _KCH)], dst_c[b], msem[b])
        pltpu.async_copy(sea_hbm.at[pl.ds(base, _KCH)], ea_c[b], msem[b])

    def meta_wait(base, b):
        pltpu.make_async_copy(ssrc_hbm.at[pl.ds(base, _KCH)], src_c[b],
                              msem[b]).wait()
        pltpu.make_async_copy(sdst_hbm.at[pl.ds(base, _KCH)], dst_c[b],
                              msem[b]).wait()
        pltpu.make_async_copy(sea_hbm.at[pl.ds(base, _KCH)], ea_c[b],
                              msem[b]).wait()

    def gather_issue(b):
        pltpu.async_copy(xl_hbm.at[src_c[b]], lrows[b], gsem[b])
        pltpu.async_copy(xr_hbm.at[dst_c[b]], rrows[b], gsem[b])

    def gather_wait(b):
        pltpu.make_async_copy(xl_hbm.at[src_c[b]], lrows[b], gsem[b]).wait()
        pltpu.make_async_copy(xr_hbm.at[dst_c[b]], rrows[b], gsem[b]).wait()

    @pl.when(nch > 0)
    def _prolog():
        meta_issue(c0, 0)
        meta_wait(c0, 0)
        gather_issue(0)

    @pl.when(nch > 1)
    def _prolog2():
        meta_issue(_bofs(1), 1)

    def pair(cp, carry):
        for b in (0, 1):
            ci = 2 * cp + b
            ob = 1 - b

            @pl.when(ci + 1 < nch)
            def _():
                meta_wait(_bofs(ci + 1), ob)
                gather_issue(ob)

            def _comp(c, b=b, ci=ci):
                gather_wait(b)
                base = _bofs(ci)
                j_lo = jnp.maximum(e_lo - base, 0)
                j_hi = jnp.minimum(e_hi - base, _KCH)
                return lax.fori_loop(j_lo, j_hi,
                                     functools.partial(process, b), c)

            carry = lax.cond(ci < nch, _comp, lambda c: c, carry)

            @pl.when(ci + 2 < nch)
            def _():
                meta_issue(_bofs(ci + 2), b)
        return carry

    carry = lax.fori_loop(0, (nch + 1) // 2, pair, (n0, aux0))
    cur, aux = finalize_to(n1, carry)
    if pool:
        off = pl.multiple_of((wid * _G + aux) * _HC, _HC)
        pltpu.sync_copy(gmax_v, out_hbm.at[pl.ds(off, _HC)])


def _run_gat_sc(pool, xl, xr, ssrc, sdst, sea, est, wev, attv, biasv, batch):
    mesh = plsc.VectorSubcoreMesh(core_axis_name="c", subcore_axis_name="s")
    out_type = (jax.ShapeDtypeStruct((_NT * _G * _HC,), jnp.float32) if pool
                else jax.ShapeDtypeStruct((_N, _HC), jnp.float32))
    scratch = [
        pltpu.VMEM((40,), jnp.int32),
        pltpu.VMEM((_HC,), jnp.float32),
        pltpu.VMEM((_HC,), jnp.float32),
        pltpu.VMEM((_HC,), jnp.float32),
        pltpu.VMEM((328,), jnp.int32),
        pltpu.VMEM((_KCH,), jnp.int32),
        pltpu.VMEM((_KCH,), jnp.int32),
        pltpu.VMEM((_KCH,), jnp.int32),
        pltpu.VMEM((_KCH,), jnp.int32),
        pltpu.VMEM((_KCH,), jnp.float32),
        pltpu.VMEM((_KCH,), jnp.float32),
        pltpu.VMEM((_KCH, _HC // 2), jnp.int32),
        pltpu.VMEM((_KCH, _HC // 2), jnp.int32),
        pltpu.VMEM((_KCH, _HC // 2), jnp.int32),
        pltpu.VMEM((_KCH, _HC // 2), jnp.int32),
        pltpu.VMEM((_HC,), jnp.float32),
        pltpu.VMEM((8, _HC), jnp.float32),
        pltpu.VMEM((_HC,), jnp.float32),
        pltpu.VMEM((_HC,), jnp.float32),
        pltpu.VMEM((64,), jnp.float32),
        pltpu.SemaphoreType.DMA,
        pltpu.SemaphoreType.DMA,
        pltpu.SemaphoreType.DMA,
        pltpu.SemaphoreType.DMA,
    ]
    k = pl.kernel(functools.partial(_gat_body, pool), mesh=mesh,
                  out_type=out_type, scratch_types=scratch)
    return k(xl, xr, ssrc, sdst, sea, est, wev, attv, biasv, batch)


def _mm_dual_body(a_ref, b_ref, bias_ref, l_ref, r_ref):
    a = a_ref[...]
    if a.dtype != jnp.bfloat16:
        a = a.astype(jnp.bfloat16)
    b = b_ref[...]
    l_ref[...] = (jnp.dot(a, b[:, :_HC], preferred_element_type=jnp.float32)
                  + bias_ref[0, :_HC]).astype(jnp.bfloat16)
    r_ref[...] = (jnp.dot(a, b[:, _HC:], preferred_element_type=jnp.float32)
                  + bias_ref[0, _HC:]).astype(jnp.bfloat16)


def _mm_dual(a, w, bias):
    m, kk = a.shape
    blk = 1000
    return pl.pallas_call(
        _mm_dual_body,
        grid=(m // blk,),
        in_specs=[pl.BlockSpec((blk, kk), lambda i: (i, 0)),
                  pl.BlockSpec((kk, 2 * _HC), lambda i: (0, 0)),
                  pl.BlockSpec((1, 2 * _HC), lambda i: (0, 0))],
        out_specs=[pl.BlockSpec((blk, _HC), lambda i: (i, 0)),
                   pl.BlockSpec((blk, _HC), lambda i: (i, 0))],
        out_shape=[jax.ShapeDtypeStruct((m, _HC), jnp.bfloat16),
                   jax.ShapeDtypeStruct((m, _HC), jnp.bfloat16)],
    )(a, w.astype(jnp.bfloat16), bias.reshape(1, -1))


def _pool_mlp_body(p_ref, wp_ref, bp_ref, w1_ref, b1_ref, w2_ref, b2_ref,
                   o_ref):
    gmax = jnp.max(p_ref[...], axis=0)
    gmax = jnp.where(gmax < -1e29, 0.0, gmax)
    p = (jnp.dot(gmax, wp_ref[...], preferred_element_type=jnp.float32)
         + bp_ref[0])
    z = jnp.maximum(
        jnp.dot(p, w1_ref[...], preferred_element_type=jnp.float32)
        + b1_ref[0], 0.0)
    o_ref[...] = (jnp.dot(z, w2_ref[...], preferred_element_type=jnp.float32)
                  + b2_ref[0])


def _pool_mlp(partials, wp, bp, w1, b1, w2, b2):
    return pl.pallas_call(
        _pool_mlp_body,
        out_shape=jax.ShapeDtypeStruct((_G, w2.shape[1]), jnp.float32),
    )(partials, wp, bp.reshape(1, -1), w1, b1.reshape(1, -1), w2,
      b2.reshape(1, -1))


def kernel(x, edge_index, edge_attr, batch, Wl1, bl1, Wr1, br1, We1, att1,
           bias1, Wl2, bl2, Wr2, br2, We2, att2, bias2, Wp, bp, W1, b1, W2,
           b2):
    src = edge_index[0].astype(jnp.int32)
    dst = edge_index[1].astype(jnp.int32)
    ea = edge_attr[:, 0]

    eid = lax.iota(jnp.uint32, _E)
    skey = lax.sort((dst.astype(jnp.uint32) << 18) | eid)
    perm = (skey & jnp.uint32(0x3FFFF)).astype(jnp.int32)
    sdst = (skey >> 18).astype(jnp.int32)
    ssrc = jnp.take(src, perm)
    sea = jnp.take(ea, perm)
    pad = _KCH + 8
    sdst_p = jnp.concatenate([sdst, jnp.zeros((pad,), jnp.int32)])
    ssrc_p = jnp.concatenate([ssrc, jnp.zeros((pad,), jnp.int32)])
    sea_p = jnp.concatenate([sea, jnp.zeros((pad,), jnp.float32)])
    iperm = []
    for g in range(_HC // 32):
        iperm += [g * 32 + 2 * i for i in range(16)]
        iperm += [g * 32 + 2 * i + 1 for i in range(16)]
    iperm = jnp.array(iperm, jnp.int32)
    bounds = jnp.array([t * _NPB for t in range(_NT)] + [_N], jnp.int32)
    est = jnp.concatenate([
        jnp.searchsorted(sdst, bounds, side='left').astype(jnp.int32),
        jnp.zeros((7,), jnp.int32)])
    batch32 = batch.astype(jnp.int32)

    def _pack_rows(m2d):
        return lax.bitcast_convert_type(
            m2d.reshape(m2d.shape[0], _HC // 2, 2), jnp.int32)

    xl1, xr1 = _mm_dual(x, jnp.concatenate([Wl1, Wr1], axis=1),
                        jnp.concatenate([bl1, br1]))
    h1 = _run_gat_sc(False, _pack_rows(xl1), _pack_rows(xr1), ssrc_p,
                     sdst_p, sea_p, est, We1[0].astype(jnp.float32)[iperm],
                     att1.reshape(-1).astype(jnp.float32)[iperm],
                     bias1[iperm], batch32)
    xl2, xr2 = _mm_dual(h1, jnp.concatenate([Wl2, Wr2], axis=1)[iperm],
                        jnp.concatenate([bl2, br2]))
    partials = _run_gat_sc(True, _pack_rows(xl2), _pack_rows(xr2), ssrc_p,
                           sdst_p, sea_p, est, We2[0].astype(jnp.float32)[iperm],
                           att2.reshape(-1).astype(jnp.float32)[iperm],
                           bias2[iperm], batch32)
    return _pool_mlp(partials.reshape(_NT, _G, _HC), Wp[iperm], bp, W1, b1,
                     W2, b2)
